# Initial kernel scaffold; baseline (speedup 1.0000x reference)
import jax
import jax.numpy as jnp
from jax.experimental import pallas as pl


def kernel(x, emb_table, lin_table, bias, bn0_g, bn0_b, W1, b1, bn1_g, bn1_b, W2, b2, bn2_g, bn2_b, W3, b3):
    raise NotImplementedError("write your pallas kernel here")


# speedup vs baseline: 1.2973x; 1.2973x over previous
import functools

import jax
import jax.numpy as jnp
from jax import lax
from jax.experimental import pallas as pl
from jax.experimental.pallas import tpu as pltpu
from jax.experimental.pallas import tpu_sc as plsc

B = 16384
F = 26
PER = 38461
TOTAL = F * PER
D = 16
H1 = 64
H2 = 32
EPS = 1e-5

NC = 2
NS = 16
NW = NC * NS
CHUNK = 128
NCHUNK = B // CHUNK
CPW = NCHUNK // NW
GSZ = F * CHUNK


def _sc_body(emb_hbm, lin_hbm, idx_hbm, cross_hbm, lsum_hbm,
             idx_v, emb_v, lin_v, out_v, lout_v, sem_e, sem_l):
    wid = lax.axis_index("s") * NC + lax.axis_index("c")
    for g in range(CPW):
        c = wid * CPW + g
        pltpu.sync_copy(idx_hbm.at[c], idx_v)
        ce = pltpu.async_copy(emb_hbm.at[idx_v], emb_v, sem_e)
        cl = pltpu.async_copy(lin_hbm.at[idx_v], lin_v, sem_l)
        ce.wait()
        cl.wait()

        def emb_row(j, _):
            e0 = emb_v[j, :]
            s = e0
            q = e0 * e0
            for f in range(1, F):
                e = emb_v[f * CHUNK + j, :]
                s = s + e
                q = q + e * e
            out_v[j, :] = s * s - q
            return 0

        lax.fori_loop(0, CHUNK, emb_row, 0, unroll=2)

        def lin_blk(t, _):
            a = lin_v[pl.ds(t * 16, 16)]
            for f in range(1, F):
                a = a + lin_v[pl.ds(f * CHUNK + t * 16, 16)]
            lout_v[pl.ds(t * 16, 16)] = a
            return 0

        lax.fori_loop(0, CHUNK // 16, lin_blk, 0)

        r0 = c * CHUNK
        pltpu.sync_copy(out_v, cross_hbm.at[pl.ds(r0, CHUNK), :])
        pltpu.sync_copy(lout_v, lsum_hbm.at[pl.ds(r0, CHUNK)])


@functools.lru_cache(maxsize=1)
def _get_fm_gather():
    return pl.kernel(
        _sc_body,
        out_type=(jax.ShapeDtypeStruct((B, D), jnp.float32),
                  jax.ShapeDtypeStruct((B,), jnp.float32)),
        mesh=plsc.VectorSubcoreMesh(core_axis_name="c", subcore_axis_name="s",
                                    num_cores=NC, num_subcores=NS),
        scratch_types=[
            pltpu.VMEM((GSZ,), jnp.int32),
            pltpu.VMEM((GSZ, D), jnp.float32),
            pltpu.VMEM((GSZ,), jnp.float32),
            pltpu.VMEM((CHUNK, D), jnp.float32),
            pltpu.VMEM((CHUNK,), jnp.float32),
            pltpu.SemaphoreType.DMA,
            pltpu.SemaphoreType.DMA,
        ],
        compiler_params=pltpu.CompilerParams(use_tc_tiling_on_sc=False),
    )


BT = 2048


def _mlp_body(raw_ref, lin_ref, w1_ref, b1_ref, w2_ref, b2_ref, w3_ref,
              b3_ref, out_ref):
    raw = raw_ref[...]
    h = jnp.dot(raw, w1_ref[...], preferred_element_type=jnp.float32)
    h = jnp.maximum(h + b1_ref[...], 0.0)
    h = jnp.dot(h, w2_ref[...], preferred_element_type=jnp.float32)
    h = jnp.maximum(h + b2_ref[...], 0.0)
    y = jnp.dot(h, w3_ref[...], preferred_element_type=jnp.float32)
    out_ref[...] = jax.nn.sigmoid(y + lin_ref[...] + b3_ref[...])


def _mlp(raw, lin2d, w1f, b1f, w2f, b2f, w3, b3f):
    grid = (B // BT,)
    return pl.pallas_call(
        _mlp_body,
        grid=grid,
        in_specs=[
            pl.BlockSpec((BT, D), lambda i: (i, 0)),
            pl.BlockSpec((BT, 1), lambda i: (i, 0)),
            pl.BlockSpec((D, H1), lambda i: (0, 0)),
            pl.BlockSpec((1, H1), lambda i: (0, 0)),
            pl.BlockSpec((H1, H2), lambda i: (0, 0)),
            pl.BlockSpec((1, H2), lambda i: (0, 0)),
            pl.BlockSpec((H2, 1), lambda i: (0, 0)),
            pl.BlockSpec((1, 1), lambda i: (0, 0)),
        ],
        out_specs=pl.BlockSpec((BT, 1), lambda i: (i, 0)),
        out_shape=jax.ShapeDtypeStruct((B, 1), jnp.float32),
    )(raw, lin2d, w1f, b1f, w2f, b2f, w3, b3f)


def kernel(x, emb_table, lin_table, bias, bn0_g, bn0_b, W1, b1, bn1_g, bn1_b,
           W2, b2, bn2_g, bn2_b, W3, b3):
    offsets = (jnp.arange(F, dtype=jnp.int32) * PER)
    idx = x.astype(jnp.int32) + offsets[None, :]
    idxc = idx.reshape(NCHUNK, CHUNK, F).transpose(0, 2, 1).reshape(NCHUNK, GSZ)
    lin_flat = lin_table.reshape(TOTAL)

    cross_raw, lsum = _get_fm_gather()(emb_table, lin_flat, idxc)

    inv = 1.0 / jnp.sqrt(jnp.float32(1.0 + EPS))
    pre_g = 0.5 * bn0_g * inv
    pre_b = bn0_b
    s1 = bn1_g * inv
    w1s = W1 * s1[None, :]
    w1f = pre_g[:, None] * w1s
    b1f = (pre_b @ w1s + b1 * s1 + bn1_b)[None, :]
    s2 = bn2_g * inv
    w2f = W2 * s2[None, :]
    b2f = (b2 * s2 + bn2_b)[None, :]
    b3f = (b3 + bias).reshape(1, 1)

    out = _mlp(cross_raw, lsum.reshape(B, 1), w1f, b1f, w2f, b2f, W3, b3f)
    return out[:, 0]


# speedup vs baseline: 3.3814x; 2.6066x over previous
import functools

import jax
import jax.numpy as jnp
from jax import lax
from jax.experimental import pallas as pl
from jax.experimental.pallas import tpu as pltpu
from jax.experimental.pallas import tpu_sc as plsc

B = 16384
F = 26
PER = 38461
TOTAL = F * PER
D = 16
H1 = 64
H2 = 32
EPS = 1e-5

NC = 2
NS = 16
NW = NC * NS
CHUNK = 128
NCHUNK = B // CHUNK
CPW = NCHUNK // NW
GSZ = F * CHUNK


def _sc_body(emb_hbm, lin_hbm, idx_hbm, cross_hbm, lsum_hbm,
             idx0, idx1, emb0, emb1, lnv0, lnv1, out0, out1, lo0, lo1,
             se0, se1, sl0, sl1, soc0, soc1, sol0, sol1):
    wid = lax.axis_index("s") * NC + lax.axis_index("c")
    idxs = (idx0, idx1)
    embs = (emb0, emb1)
    lins = (lnv0, lnv1)
    outs = (out0, out1)
    louts = (lo0, lo1)
    ses = (se0, se1)
    sls = (sl0, sl1)
    socs = (soc0, soc1)
    sols = (sol0, sol1)

    def fire_gather(g, b):
        pltpu.sync_copy(idx_hbm.at[wid * CPW + g], idxs[b])
        pltpu.async_copy(emb_hbm.at[idxs[b]], embs[b], ses[b])
        pltpu.async_copy(lin_hbm.at[idxs[b]], lins[b], sls[b])

    def wait_gather(b):
        pltpu.make_async_copy(emb_hbm.at[idxs[b]], embs[b], ses[b]).wait()
        pltpu.make_async_copy(lin_hbm.at[idxs[b]], lins[b], sls[b]).wait()

    fire_gather(0, 0)
    for g in range(CPW):
        b = g & 1
        if g + 1 < CPW:
            fire_gather(g + 1, 1 - b)
        wait_gather(b)
        if g >= 2:
            pltpu.make_async_copy(outs[b], cross_hbm.at[pl.ds(0, CHUNK), :],
                                  socs[b]).wait()
            pltpu.make_async_copy(louts[b], lsum_hbm.at[pl.ds(0, CHUNK)],
                                  sols[b]).wait()
        emb_v, lin_v, out_v, lout_v = embs[b], lins[b], outs[b], louts[b]

        def emb_row(j, _):
            e0 = emb_v[j, :]
            s = e0
            q = e0 * e0
            for f in range(1, F):
                e = emb_v[f * CHUNK + j, :]
                s = s + e
                q = q + e * e
            out_v[j, :] = s * s - q
            return 0

        lax.fori_loop(0, CHUNK, emb_row, 0, unroll=2)

        def lin_blk(t, _):
            a = lin_v[pl.ds(t * 16, 16)]
            for f in range(1, F):
                a = a + lin_v[pl.ds(f * CHUNK + t * 16, 16)]
            lout_v[pl.ds(t * 16, 16)] = a
            return 0

        lax.fori_loop(0, CHUNK // 16, lin_blk, 0)

        r0 = (wid * CPW + g) * CHUNK
        pltpu.async_copy(out_v, cross_hbm.at[pl.ds(r0, CHUNK), :], socs[b])
        pltpu.async_copy(lout_v, lsum_hbm.at[pl.ds(r0, CHUNK)], sols[b])
    for b in range(2):
        pltpu.make_async_copy(outs[b], cross_hbm.at[pl.ds(0, CHUNK), :],
                              socs[b]).wait()
        pltpu.make_async_copy(louts[b], lsum_hbm.at[pl.ds(0, CHUNK)],
                              sols[b]).wait()


@functools.lru_cache(maxsize=1)
def _get_fm_gather():
    return pl.kernel(
        _sc_body,
        out_type=(jax.ShapeDtypeStruct((B, D), jnp.float32),
                  jax.ShapeDtypeStruct((B,), jnp.float32)),
        name="fm_gather",
        mesh=plsc.VectorSubcoreMesh(core_axis_name="c", subcore_axis_name="s",
                                    num_cores=NC, num_subcores=NS),
        scratch_types=(
            [pltpu.VMEM((GSZ,), jnp.int32)] * 2
            + [pltpu.VMEM((GSZ, D), jnp.float32)] * 2
            + [pltpu.VMEM((GSZ,), jnp.float32)] * 2
            + [pltpu.VMEM((CHUNK, D), jnp.float32)] * 2
            + [pltpu.VMEM((CHUNK,), jnp.float32)] * 2
            + [pltpu.SemaphoreType.DMA] * 8
        ),
        compiler_params=pltpu.CompilerParams(use_tc_tiling_on_sc=False),
    )


V = 1000000
CC = 512
NFC = TOTAL // CC
TAIL0 = NFC * CC
TAILN = TOTAL - TAIL0


NJC = 1952 // NW


def _conv_sc_body(et_hbm, tail_hbm, out_hbm,
                  tile0, tile1, out0, out1, tail_v,
                  sin0, sin1, sout0, sout1):
    wid = lax.axis_index("s") * NC + lax.axis_index("c")
    c0w = wid * NJC
    base_idx = lax.iota(jnp.int32, 16) * D
    tiles = (tile0, tile1)
    outs = (out0, out1)
    sins = (sin0, sin1)
    souts = (sout0, sout1)

    def convert(tv, ov):
        def c0loop(c0, _):
            for d in range(D):
                v = tv[d, pl.ds(c0 * 16, 16)]
                plsc.store_scatter(ov, [base_idx + (c0 * 16 * D + d)], v)
            return 0
        lax.fori_loop(0, CC // 16, c0loop, 0, unroll=4)

    def fire_in(j, b):
        pltpu.async_copy(et_hbm.at[:, pl.ds((c0w + j) * CC, CC)], tiles[b],
                         sins[b])

    def wait_in(b):
        pltpu.make_async_copy(et_hbm.at[:, pl.ds(0, CC)], tiles[b],
                              sins[b]).wait()

    def fire_out(j, b):
        pltpu.async_copy(outs[b], out_hbm.at[pl.ds((c0w + j) * CC * D, CC * D)],
                         souts[b])

    def wait_out(b):
        pltpu.make_async_copy(outs[b], out_hbm.at[pl.ds(0, CC * D)],
                              souts[b]).wait()

    fire_in(0, 0)
    fire_in(1, 1)

    def step(jj, _):
        for b in range(2):
            j = 2 * jj + b
            wait_in(b)

            @pl.when(jj > 0)
            def _():
                wait_out(b)

            convert(tiles[b], outs[b])
            fire_out(j, b)
            if b == 0:
                fire_in(j + 2, 0)
            else:
                @pl.when(jj < NJC // 2 - 1)
                def _():
                    fire_in(j + 2, 1)
        return 0

    lax.fori_loop(0, NJC // 2, step, 0)
    wait_in(0)
    wait_out(0)
    convert(tile0, out0)
    fire_out(NJC - 1, 0)
    wait_out(0)
    wait_out(1)

    @pl.when(wid == 0)
    def _():
        pltpu.sync_copy(et_hbm.at[:, pl.ds(1952 * CC, CC)], tile0)
        convert(tile0, out0)
        pltpu.sync_copy(out0, out_hbm.at[pl.ds(1952 * CC * D, CC * D)])
        pltpu.sync_copy(tail_hbm, tail_v)
        pltpu.sync_copy(tail_v, out_hbm.at[pl.ds(TAIL0 * D, TAILN * D)])


@functools.lru_cache(maxsize=1)
def _get_conv_sc():
    return pl.kernel(
        _conv_sc_body,
        out_type=jax.ShapeDtypeStruct((V * D,), jnp.float32),
        mesh=plsc.VectorSubcoreMesh(core_axis_name="c", subcore_axis_name="s",
                                    num_cores=NC, num_subcores=NS),
        scratch_types=[
            pltpu.VMEM((D, CC), jnp.float32),
            pltpu.VMEM((D, CC), jnp.float32),
            pltpu.VMEM((CC * D,), jnp.float32),
            pltpu.VMEM((CC * D,), jnp.float32),
            pltpu.VMEM((TAILN * D,), jnp.float32),
            pltpu.SemaphoreType.DMA,
            pltpu.SemaphoreType.DMA,
            pltpu.SemaphoreType.DMA,
            pltpu.SemaphoreType.DMA,
        ],
        name="conv_table",
        compiler_params=pltpu.CompilerParams(use_tc_tiling_on_sc=True,
                                             needs_layout_passes=False),
    )


BT = 2048


def _mlp_body(raw_ref, lin_ref, w1_ref, b1_ref, w2_ref, b2_ref, w3_ref,
              b3_ref, out_ref):
    raw = raw_ref[...]
    h = jnp.dot(raw, w1_ref[...], preferred_element_type=jnp.float32)
    h = jnp.maximum(h + b1_ref[...], 0.0)
    h = jnp.dot(h, w2_ref[...], preferred_element_type=jnp.float32)
    h = jnp.maximum(h + b2_ref[...], 0.0)
    y = jnp.dot(h, w3_ref[...], preferred_element_type=jnp.float32)
    out_ref[...] = jax.nn.sigmoid(y + lin_ref[...] + b3_ref[...])


def _mlp(raw, lin2d, w1f, b1f, w2f, b2f, w3, b3f):
    grid = (B // BT,)
    return pl.pallas_call(
        _mlp_body,
        grid=grid,
        in_specs=[
            pl.BlockSpec((BT, D), lambda i: (i, 0)),
            pl.BlockSpec((BT, 1), lambda i: (i, 0)),
            pl.BlockSpec((D, H1), lambda i: (0, 0)),
            pl.BlockSpec((1, H1), lambda i: (0, 0)),
            pl.BlockSpec((H1, H2), lambda i: (0, 0)),
            pl.BlockSpec((1, H2), lambda i: (0, 0)),
            pl.BlockSpec((H2, 1), lambda i: (0, 0)),
            pl.BlockSpec((1, 1), lambda i: (0, 0)),
        ],
        out_specs=pl.BlockSpec((BT, 1), lambda i: (i, 0)),
        out_shape=jax.ShapeDtypeStruct((B, 1), jnp.float32),
    )(raw, lin2d, w1f, b1f, w2f, b2f, w3, b3f)


def kernel(x, emb_table, lin_table, bias, bn0_g, bn0_b, W1, b1, bn1_g, bn1_b,
           W2, b2, bn2_g, bn2_b, W3, b3):
    offsets = (jnp.arange(F, dtype=jnp.int32) * PER)
    idx = x.astype(jnp.int32) + offsets[None, :]
    idxc = idx.reshape(NCHUNK, CHUNK, F).transpose(0, 2, 1).reshape(NCHUNK, GSZ)
    lin_flat = lin_table.reshape(TOTAL)

    tail_lin = emb_table[TAIL0:].reshape(TAILN * D)
    emb_lin = _get_conv_sc()(emb_table.T, tail_lin).reshape(V, D)
    cross_raw, lsum = _get_fm_gather()(emb_lin, lin_flat, idxc)

    inv = 1.0 / jnp.sqrt(jnp.float32(1.0 + EPS))
    pre_g = 0.5 * bn0_g * inv
    pre_b = bn0_b
    s1 = bn1_g * inv
    w1s = W1 * s1[None, :]
    w1f = pre_g[:, None] * w1s
    b1f = (pre_b @ w1s + b1 * s1 + bn1_b)[None, :]
    s2 = bn2_g * inv
    w2f = W2 * s2[None, :]
    b2f = (b2 * s2 + bn2_b)[None, :]
    b3f = (b3 + bias).reshape(1, 1)

    out = _mlp(cross_raw, lsum.reshape(B, 1), w1f, b1f, w2f, b2f, W3, b3f)
    return out[:, 0]


# speedup vs baseline: 3.4198x; 1.0113x over previous
import functools

import jax
import jax.numpy as jnp
from jax import lax
from jax.experimental import pallas as pl
from jax.experimental.pallas import tpu as pltpu
from jax.experimental.pallas import tpu_sc as plsc

B = 16384
F = 26
PER = 38461
TOTAL = F * PER
D = 16
H1 = 64
H2 = 32
EPS = 1e-5

NC = 2
NS = 16
NW = NC * NS
CHUNK = 128
NCHUNK = B // CHUNK
CPW = NCHUNK // NW
GSZ = F * CHUNK


def _sc_body(emb_hbm, lin_hbm, idx_hbm, cross_hbm, lsum_hbm,
             idx0, idx1, emb0, emb1, lnv0, lnv1, out0, out1, lo0, lo1,
             se0, se1, sl0, sl1, soc0, soc1, sol0, sol1):
    wid = lax.axis_index("s") * NC + lax.axis_index("c")
    idxs = (idx0, idx1)
    embs = (emb0, emb1)
    lins = (lnv0, lnv1)
    outs = (out0, out1)
    louts = (lo0, lo1)
    ses = (se0, se1)
    sls = (sl0, sl1)
    socs = (soc0, soc1)
    sols = (sol0, sol1)

    def fire_gather(g, b):
        pltpu.sync_copy(idx_hbm.at[wid * CPW + g], idxs[b])
        pltpu.async_copy(emb_hbm.at[idxs[b]], embs[b], ses[b])
        pltpu.async_copy(lin_hbm.at[idxs[b]], lins[b], sls[b])

    def wait_gather(b):
        pltpu.make_async_copy(emb_hbm.at[idxs[b]], embs[b], ses[b]).wait()
        pltpu.make_async_copy(lin_hbm.at[idxs[b]], lins[b], sls[b]).wait()

    fire_gather(0, 0)
    for g in range(CPW):
        b = g & 1
        if g + 1 < CPW:
            fire_gather(g + 1, 1 - b)
        wait_gather(b)
        if g >= 2:
            pltpu.make_async_copy(outs[b], cross_hbm.at[pl.ds(0, CHUNK), :],
                                  socs[b]).wait()
            pltpu.make_async_copy(louts[b], lsum_hbm.at[pl.ds(0, CHUNK)],
                                  sols[b]).wait()
        emb_v, lin_v, out_v, lout_v = embs[b], lins[b], outs[b], louts[b]

        def emb_row(j, _):
            e0 = emb_v[j, :]
            s = e0
            q = e0 * e0
            for f in range(1, F):
                e = emb_v[f * CHUNK + j, :]
                s = s + e
                q = q + e * e
            out_v[j, :] = s * s - q
            return 0

        lax.fori_loop(0, CHUNK, emb_row, 0, unroll=2)

        def lin_blk(t, _):
            a = lin_v[pl.ds(t * 16, 16)]
            for f in range(1, F):
                a = a + lin_v[pl.ds(f * CHUNK + t * 16, 16)]
            lout_v[pl.ds(t * 16, 16)] = a
            return 0

        lax.fori_loop(0, CHUNK // 16, lin_blk, 0)

        r0 = (wid * CPW + g) * CHUNK
        pltpu.async_copy(out_v, cross_hbm.at[pl.ds(r0, CHUNK), :], socs[b])
        pltpu.async_copy(lout_v, lsum_hbm.at[pl.ds(r0, CHUNK)], sols[b])
    for b in range(2):
        pltpu.make_async_copy(outs[b], cross_hbm.at[pl.ds(0, CHUNK), :],
                              socs[b]).wait()
        pltpu.make_async_copy(louts[b], lsum_hbm.at[pl.ds(0, CHUNK)],
                              sols[b]).wait()


@functools.lru_cache(maxsize=1)
def _get_fm_gather():
    return pl.kernel(
        _sc_body,
        out_type=(jax.ShapeDtypeStruct((B, D), jnp.float32),
                  jax.ShapeDtypeStruct((B,), jnp.float32)),
        name="fm_gather",
        mesh=plsc.VectorSubcoreMesh(core_axis_name="c", subcore_axis_name="s",
                                    num_cores=NC, num_subcores=NS),
        scratch_types=(
            [pltpu.VMEM((GSZ,), jnp.int32)] * 2
            + [pltpu.VMEM((GSZ, D), jnp.float32)] * 2
            + [pltpu.VMEM((GSZ,), jnp.float32)] * 2
            + [pltpu.VMEM((CHUNK, D), jnp.float32)] * 2
            + [pltpu.VMEM((CHUNK,), jnp.float32)] * 2
            + [pltpu.SemaphoreType.DMA] * 8
        ),
        compiler_params=pltpu.CompilerParams(use_tc_tiling_on_sc=False),
    )


V = 1000000
CC = 512
NFC = TOTAL // CC
TAIL0 = NFC * CC
TAILN = TOTAL - TAIL0


NJC = 1952 // NW


def _conv_sc_body(et_hbm, tail_hbm, out_hbm,
                  tile0, tile1, out0, out1, tail_v,
                  sin0, sin1, sout0, sout1):
    wid = lax.axis_index("s") * NC + lax.axis_index("c")
    c0w = wid * NJC
    base_idx = lax.iota(jnp.int32, 16) * D
    tiles = (tile0, tile1)
    outs = (out0, out1)
    sins = (sin0, sin1)
    souts = (sout0, sout1)

    def convert(tv, ov):
        def c0loop(c0, _):
            for d in range(D):
                v = tv[d, pl.ds(c0 * 16, 16)]
                plsc.store_scatter(ov, [base_idx + (c0 * 16 * D + d)], v)
            return 0
        lax.fori_loop(0, CC // 16, c0loop, 0)

    def fire_in(j, b):
        pltpu.async_copy(et_hbm.at[:, pl.ds((c0w + j) * CC, CC)], tiles[b],
                         sins[b])

    def wait_in(b):
        pltpu.make_async_copy(et_hbm.at[:, pl.ds(0, CC)], tiles[b],
                              sins[b]).wait()

    def fire_out(j, b):
        pltpu.async_copy(outs[b], out_hbm.at[pl.ds((c0w + j) * CC * D, CC * D)],
                         souts[b])

    def wait_out(b):
        pltpu.make_async_copy(outs[b], out_hbm.at[pl.ds(0, CC * D)],
                              souts[b]).wait()

    fire_in(0, 0)
    fire_in(1, 1)

    def step(jj, _):
        for b in range(2):
            j = 2 * jj + b
            wait_in(b)

            @pl.when(jj > 0)
            def _():
                wait_out(b)

            convert(tiles[b], outs[b])
            fire_out(j, b)
            if b == 0:
                fire_in(j + 2, 0)
            else:
                @pl.when(jj < NJC // 2 - 1)
                def _():
                    fire_in(j + 2, 1)
        return 0

    lax.fori_loop(0, NJC // 2, step, 0)
    wait_in(0)
    wait_out(0)
    convert(tile0, out0)
    fire_out(NJC - 1, 0)
    wait_out(0)
    wait_out(1)

    @pl.when(wid == 0)
    def _():
        pltpu.sync_copy(et_hbm.at[:, pl.ds(1952 * CC, CC)], tile0)
        convert(tile0, out0)
        pltpu.sync_copy(out0, out_hbm.at[pl.ds(1952 * CC * D, CC * D)])
        pltpu.sync_copy(tail_hbm, tail_v)
        pltpu.sync_copy(tail_v, out_hbm.at[pl.ds(TAIL0 * D, TAILN * D)])


@functools.lru_cache(maxsize=1)
def _get_conv_sc():
    return pl.kernel(
        _conv_sc_body,
        out_type=jax.ShapeDtypeStruct((V * D,), jnp.float32),
        mesh=plsc.VectorSubcoreMesh(core_axis_name="c", subcore_axis_name="s",
                                    num_cores=NC, num_subcores=NS),
        scratch_types=[
            pltpu.VMEM((D, CC), jnp.float32),
            pltpu.VMEM((D, CC), jnp.float32),
            pltpu.VMEM((CC * D,), jnp.float32),
            pltpu.VMEM((CC * D,), jnp.float32),
            pltpu.VMEM((TAILN * D,), jnp.float32),
            pltpu.SemaphoreType.DMA,
            pltpu.SemaphoreType.DMA,
            pltpu.SemaphoreType.DMA,
            pltpu.SemaphoreType.DMA,
        ],
        name="conv_table",
        compiler_params=pltpu.CompilerParams(use_tc_tiling_on_sc=True,
                                             needs_layout_passes=False),
    )


BT = 2048


def _mlp_body(raw_ref, lin_ref, w1_ref, b1_ref, w2_ref, b2_ref, w3_ref,
              b3_ref, out_ref):
    raw = raw_ref[...]
    h = jnp.dot(raw, w1_ref[...], preferred_element_type=jnp.float32)
    h = jnp.maximum(h + b1_ref[...], 0.0)
    h = jnp.dot(h, w2_ref[...], preferred_element_type=jnp.float32)
    h = jnp.maximum(h + b2_ref[...], 0.0)
    y = jnp.dot(h, w3_ref[...], preferred_element_type=jnp.float32)
    out_ref[...] = jax.nn.sigmoid(y + lin_ref[...] + b3_ref[...])


def _mlp(raw, lin2d, w1f, b1f, w2f, b2f, w3, b3f):
    grid = (B // BT,)
    return pl.pallas_call(
        _mlp_body,
        grid=grid,
        in_specs=[
            pl.BlockSpec((BT, D), lambda i: (i, 0)),
            pl.BlockSpec((BT, 1), lambda i: (i, 0)),
            pl.BlockSpec((D, H1), lambda i: (0, 0)),
            pl.BlockSpec((1, H1), lambda i: (0, 0)),
            pl.BlockSpec((H1, H2), lambda i: (0, 0)),
            pl.BlockSpec((1, H2), lambda i: (0, 0)),
            pl.BlockSpec((H2, 1), lambda i: (0, 0)),
            pl.BlockSpec((1, 1), lambda i: (0, 0)),
        ],
        out_specs=pl.BlockSpec((BT, 1), lambda i: (i, 0)),
        out_shape=jax.ShapeDtypeStruct((B, 1), jnp.float32),
    )(raw, lin2d, w1f, b1f, w2f, b2f, w3, b3f)


def kernel(x, emb_table, lin_table, bias, bn0_g, bn0_b, W1, b1, bn1_g, bn1_b,
           W2, b2, bn2_g, bn2_b, W3, b3):
    offsets = (jnp.arange(F, dtype=jnp.int32) * PER)
    idx = x.astype(jnp.int32) + offsets[None, :]
    idxc = idx.reshape(NCHUNK, CHUNK, F).transpose(0, 2, 1).reshape(NCHUNK, GSZ)
    lin_flat = lin_table.reshape(TOTAL)

    tail_lin = emb_table[TAIL0:].reshape(TAILN * D)
    emb_lin = _get_conv_sc()(emb_table.T, tail_lin).reshape(V, D)
    cross_raw, lsum = _get_fm_gather()(emb_lin, lin_flat, idxc)

    inv = 1.0 / jnp.sqrt(jnp.float32(1.0 + EPS))
    pre_g = 0.5 * bn0_g * inv
    pre_b = bn0_b
    s1 = bn1_g * inv
    w1s = W1 * s1[None, :]
    w1f = pre_g[:, None] * w1s
    b1f = (pre_b @ w1s + b1 * s1 + bn1_b)[None, :]
    s2 = bn2_g * inv
    w2f = W2 * s2[None, :]
    b2f = (b2 * s2 + bn2_b)[None, :]
    b3f = (b3 + bias).reshape(1, 1)

    out = _mlp(cross_raw, lsum.reshape(B, 1), w1f, b1f, w2f, b2f, W3, b3f)
    return out[:, 0]
